# baseline (device time: 105006 ns/iter reference)
import jax
import jax.numpy as jnp
import numpy as np
from jax import lax
from jax.experimental import pallas as pl
from jax.experimental.pallas import tpu as pltpu

N_DEV = 4
B = 2
SQ = 512
SKV = 512
HQ = 32
DH = 64
HG = HQ // N_DEV
D_MODEL = 768
D_HEADS = HQ * DH
G_COLS = HG * DH
QB = 64
NB = SQ // QB
R = 4
RROWS = SQ // R

PERM = [0, 4, 1, 5, 2, 6, 3, 7]
POS = [PERM.index(i) for i in range(NB)]


def kernel(x, Wq, K_ext, V_ext, Wo):
    bf = jnp.bfloat16
    perm = np.array(PERM)

    def permute_rows(a):
        return a.reshape(B, NB, QB, -1)[:, perm].reshape(B, SQ, -1)

    K2 = permute_rows(K_ext.reshape(B, SKV, D_HEADS)).astype(bf)
    V2 = permute_rows(V_ext.reshape(B, SKV, D_HEADS)).astype(bf)
    xp = permute_rows(x).astype(bf)
    Wq = Wq.astype(bf)
    Wo = Wo.astype(bf)

    def body(x_ref, wq_ref, k_ref, v_ref, wo_ref, out_ref,
             wqb, wob, qsend, qrecv, osend, orecv, csem):
        me = lax.axis_index("i")
        left = (me - 1) % N_DEV
        right = (me + 1) % N_DEV

        bsem = pltpu.get_barrier_semaphore()
        pl.semaphore_signal(bsem, inc=1, device_id=(left,),
                            device_id_type=pl.DeviceIdType.MESH)
        pl.semaphore_signal(bsem, inc=1, device_id=(right,),
                            device_id_type=pl.DeviceIdType.MESH)
        pl.semaphore_wait(bsem, 2)

        cq = pltpu.make_async_copy(wq_ref, wqb.at[0], csem.at[0])
        co = pltpu.make_async_copy(wo_ref, wob.at[0], csem.at[1])
        cq.start()
        co.start()
        cq.wait()
        co.wait()

        li = lax.broadcasted_iota(jnp.int32, (SQ, SKV), 0)
        kj = lax.broadcasted_iota(jnp.int32, (SQ, SKV), 1)
        mask = (li // RROWS) == (kj // RROWS)

        xs = [x_ref[b] for b in range(B)]
        accs = [jnp.zeros((SQ, D_MODEL), jnp.float32) for _ in range(B)]

        def compute_group(slot, origin):
            col0 = origin * G_COLS
            for b in range(B):
                q = lax.dot_general(
                    xs[b], wqb[slot], (((1,), (0,)), ((), ())),
                    preferred_element_type=jnp.float32)
                q = (q * 0.125).astype(jnp.bfloat16)
                kg = k_ref[b, :, pl.ds(col0, G_COLS)]
                vg = v_ref[b, :, pl.ds(col0, G_COLS)]
                ctx_parts = []
                for hh in range(HG):
                    qh = q[:, hh * DH:(hh + 1) * DH]
                    kh = kg[:, hh * DH:(hh + 1) * DH]
                    vh = vg[:, hh * DH:(hh + 1) * DH]
                    s = lax.dot_general(
                        qh, kh, (((1,), (1,)), ((), ())),
                        preferred_element_type=jnp.float32)
                    e = jnp.where(mask, jnp.exp(s), 0.0)
                    w = (e / jnp.sum(e, axis=1, keepdims=True)).astype(
                        jnp.bfloat16)
                    ctx_parts.append(lax.dot_general(
                        w, vh, (((1,), (0,)), ((), ())),
                        preferred_element_type=jnp.float32).astype(
                            jnp.bfloat16))
                ctx = jnp.concatenate(ctx_parts, axis=1)
                accs[b] = accs[b] + lax.dot_general(
                    ctx, wob[slot], (((1,), (0,)), ((), ())),
                    preferred_element_type=jnp.float32)

        for h in range(N_DEV - 1):
            rq = pltpu.make_async_remote_copy(
                src_ref=wqb.at[h], dst_ref=wqb.at[h + 1],
                send_sem=qsend.at[h], recv_sem=qrecv.at[h],
                device_id=(right,), device_id_type=pl.DeviceIdType.MESH)
            ro = pltpu.make_async_remote_copy(
                src_ref=wob.at[h], dst_ref=wob.at[h + 1],
                send_sem=osend.at[h], recv_sem=orecv.at[h],
                device_id=(right,), device_id_type=pl.DeviceIdType.MESH)
            rq.start()
            ro.start()
            compute_group(h, (me - h) % N_DEV)
            rq.wait()
            ro.wait()
        compute_group(N_DEV - 1, (me - (N_DEV - 1)) % N_DEV)

        for b in range(B):
            out_ref[b] = jnp.concatenate(
                [accs[b][POS[lb] * QB:(POS[lb] + 1) * QB, :]
                 for lb in range(NB)], axis=0)

    return pl.pallas_call(
        body,
        out_shape=jax.ShapeDtypeStruct((B, SQ, D_MODEL), jnp.float32),
        in_specs=[pl.BlockSpec(memory_space=pltpu.VMEM)] * 5,
        out_specs=pl.BlockSpec(memory_space=pltpu.VMEM),
        scratch_shapes=[
            pltpu.VMEM((N_DEV, D_MODEL, G_COLS), jnp.bfloat16),
            pltpu.VMEM((N_DEV, G_COLS, D_MODEL), jnp.bfloat16),
            pltpu.SemaphoreType.DMA((N_DEV - 1,)),
            pltpu.SemaphoreType.DMA((N_DEV - 1,)),
            pltpu.SemaphoreType.DMA((N_DEV - 1,)),
            pltpu.SemaphoreType.DMA((N_DEV - 1,)),
            pltpu.SemaphoreType.DMA((2,)),
        ],
        compiler_params=pltpu.CompilerParams(collective_id=0),
    )(xp, Wq, K2, V2, Wo)


# device time: 83338 ns/iter; 1.2600x vs baseline; 1.2600x over previous
import jax
import jax.numpy as jnp
from jax import lax
from jax.experimental import pallas as pl
from jax.experimental.pallas import tpu as pltpu

N_DEV = 4
B = 2
SQ = 512
SKV = 512
HQ = 32
DH = 64
HG = HQ // N_DEV
D_MODEL = 768
D_HEADS = HQ * DH
G_COLS = HG * DH


def kernel(x, Wq, K_ext, V_ext, Wo):
    bf = jnp.bfloat16
    K2 = K_ext.reshape(B, SKV, D_HEADS).astype(bf)
    V2 = V_ext.reshape(B, SKV, D_HEADS).astype(bf)
    x = x.astype(bf)
    Wq = Wq.astype(bf)
    Wo = Wo.astype(bf)

    def body(x_ref, wq_ref, k_ref, v_ref, wo_ref, out_ref,
             wqb, wob, qsend, qrecv, osend, orecv, csem):
        me = lax.axis_index("i")
        left = (me - 1) % N_DEV
        right = (me + 1) % N_DEV

        bsem = pltpu.get_barrier_semaphore()
        pl.semaphore_signal(bsem, inc=1, device_id=(left,),
                            device_id_type=pl.DeviceIdType.MESH)
        pl.semaphore_signal(bsem, inc=1, device_id=(right,),
                            device_id_type=pl.DeviceIdType.MESH)
        pl.semaphore_wait(bsem, 2)

        cq = pltpu.make_async_copy(wq_ref, wqb.at[0], csem.at[0])
        co = pltpu.make_async_copy(wo_ref, wob.at[0], csem.at[1])
        cq.start()
        co.start()
        cq.wait()
        co.wait()

        li = lax.broadcasted_iota(jnp.int32, (SQ, SKV), 0)
        kj = lax.broadcasted_iota(jnp.int32, (SQ, SKV), 1)
        mask = (li // 64) % 4 == (kj // 64) % 4

        xs = [x_ref[b] for b in range(B)]
        accs = [jnp.zeros((SQ, D_MODEL), jnp.float32) for _ in range(B)]

        def compute_group(slot, origin):
            col0 = origin * G_COLS
            for b in range(B):
                q = lax.dot_general(
                    xs[b], wqb[slot], (((1,), (0,)), ((), ())),
                    preferred_element_type=jnp.float32)
                q = (q * 0.125).astype(jnp.bfloat16)
                kg = k_ref[b, :, pl.ds(col0, G_COLS)]
                vg = v_ref[b, :, pl.ds(col0, G_COLS)]
                ctx_parts = []
                for hh in range(HG):
                    qh = q[:, hh * DH:(hh + 1) * DH]
                    kh = kg[:, hh * DH:(hh + 1) * DH]
                    vh = vg[:, hh * DH:(hh + 1) * DH]
                    s = lax.dot_general(
                        qh, kh, (((1,), (1,)), ((), ())),
                        preferred_element_type=jnp.float32)
                    e = jnp.where(mask, jnp.exp(s), 0.0)
                    w = (e / jnp.sum(e, axis=1, keepdims=True)).astype(
                        jnp.bfloat16)
                    ctx_parts.append(lax.dot_general(
                        w, vh, (((1,), (0,)), ((), ())),
                        preferred_element_type=jnp.float32).astype(
                            jnp.bfloat16))
                ctx = jnp.concatenate(ctx_parts, axis=1)
                accs[b] = accs[b] + lax.dot_general(
                    ctx, wob[slot], (((1,), (0,)), ((), ())),
                    preferred_element_type=jnp.float32)

        for h in range(N_DEV - 1):
            rq = pltpu.make_async_remote_copy(
                src_ref=wqb.at[h], dst_ref=wqb.at[h + 1],
                send_sem=qsend.at[h], recv_sem=qrecv.at[h],
                device_id=(right,), device_id_type=pl.DeviceIdType.MESH)
            ro = pltpu.make_async_remote_copy(
                src_ref=wob.at[h], dst_ref=wob.at[h + 1],
                send_sem=osend.at[h], recv_sem=orecv.at[h],
                device_id=(right,), device_id_type=pl.DeviceIdType.MESH)
            rq.start()
            ro.start()
            compute_group(h, (me - h) % N_DEV)
            rq.wait()
            ro.wait()
        compute_group(N_DEV - 1, (me - (N_DEV - 1)) % N_DEV)

        for b in range(B):
            out_ref[b] = accs[b]

    return pl.pallas_call(
        body,
        out_shape=jax.ShapeDtypeStruct((B, SQ, D_MODEL), jnp.float32),
        in_specs=[pl.BlockSpec(memory_space=pltpu.VMEM)] * 5,
        out_specs=pl.BlockSpec(memory_space=pltpu.VMEM),
        scratch_shapes=[
            pltpu.VMEM((N_DEV, D_MODEL, G_COLS), jnp.bfloat16),
            pltpu.VMEM((N_DEV, G_COLS, D_MODEL), jnp.bfloat16),
            pltpu.SemaphoreType.DMA((N_DEV - 1,)),
            pltpu.SemaphoreType.DMA((N_DEV - 1,)),
            pltpu.SemaphoreType.DMA((N_DEV - 1,)),
            pltpu.SemaphoreType.DMA((N_DEV - 1,)),
            pltpu.SemaphoreType.DMA((2,)),
        ],
        compiler_params=pltpu.CompilerParams(collective_id=0),
    )(x, Wq, K2, V2, Wo)


# device time: 65937 ns/iter; 1.5925x vs baseline; 1.2639x over previous
import jax
import jax.numpy as jnp
from jax import lax
from jax.experimental import pallas as pl
from jax.experimental.pallas import tpu as pltpu

N_DEV = 4
B = 2
SQ = 512
SKV = 512
HQ = 32
DH = 64
HG = HQ // N_DEV
D_MODEL = 768
D_HEADS = HQ * DH
G_COLS = HG * DH


def kernel(x, Wq, K_ext, V_ext, Wo):
    bf = jnp.bfloat16
    K2 = K_ext.reshape(B, SKV, D_HEADS).astype(bf)
    V2 = V_ext.reshape(B, SKV, D_HEADS).astype(bf)
    x = x.astype(bf)
    Wq = Wq.astype(bf)
    Wo = Wo.astype(bf)

    def body(x_ref, wq_ref, k_ref, v_ref, wo_ref, out_ref,
             wqb, wob, qsend, qrecv, osend, orecv, csem):
        me = lax.axis_index("i")
        left = (me - 1) % N_DEV
        right = (me + 1) % N_DEV

        bsem = pltpu.get_barrier_semaphore()
        pl.semaphore_signal(bsem, inc=1, device_id=(left,),
                            device_id_type=pl.DeviceIdType.MESH)
        pl.semaphore_signal(bsem, inc=1, device_id=(right,),
                            device_id_type=pl.DeviceIdType.MESH)
        pl.semaphore_wait(bsem, 2)

        cq = pltpu.make_async_copy(wq_ref, wqb.at[0], csem.at[0])
        co = pltpu.make_async_copy(wo_ref, wob.at[0], csem.at[1])
        cq.start()
        co.start()
        cq.wait()
        co.wait()

        li = lax.broadcasted_iota(jnp.int32, (SQ, SKV), 0)
        kj = lax.broadcasted_iota(jnp.int32, (SQ, SKV), 1)
        mask = (li // 64) % 4 == (kj // 64) % 4

        xs = [x_ref[b] for b in range(B)]
        accs = [jnp.zeros((SQ, D_MODEL), jnp.float32) for _ in range(B)]

        def compute_group(slot, origin):
            col0 = origin * G_COLS
            for b in range(B):
                q = lax.dot_general(
                    xs[b], wqb[slot], (((1,), (0,)), ((), ())),
                    preferred_element_type=jnp.float32)
                q = (q * 0.125).astype(jnp.bfloat16)
                kg = k_ref[b, :, pl.ds(col0, G_COLS)]
                vg = v_ref[b, :, pl.ds(col0, G_COLS)]
                ctx_parts = []
                for hh in range(HG):
                    qh = q[:, hh * DH:(hh + 1) * DH]
                    kh = kg[:, hh * DH:(hh + 1) * DH]
                    vh = vg[:, hh * DH:(hh + 1) * DH]
                    s = lax.dot_general(
                        qh, kh, (((1,), (1,)), ((), ())),
                        preferred_element_type=jnp.float32)
                    e = jnp.where(mask, jnp.exp(s), 0.0)
                    w = (e / jnp.sum(e, axis=1, keepdims=True)).astype(
                        jnp.bfloat16)
                    ctx_parts.append(lax.dot_general(
                        w, vh, (((1,), (0,)), ((), ())),
                        preferred_element_type=jnp.float32).astype(
                            jnp.bfloat16))
                ctx = jnp.concatenate(ctx_parts, axis=1)
                accs[b] = accs[b] + lax.dot_general(
                    ctx, wob[slot], (((1,), (0,)), ((), ())),
                    preferred_element_type=jnp.float32)

        def pair_copy(dst_dev, src_slot, dst_slot, idx):
            rq = pltpu.make_async_remote_copy(
                src_ref=wqb.at[src_slot], dst_ref=wqb.at[dst_slot],
                send_sem=qsend.at[idx], recv_sem=qrecv.at[idx],
                device_id=(dst_dev,), device_id_type=pl.DeviceIdType.MESH)
            ro = pltpu.make_async_remote_copy(
                src_ref=wob.at[src_slot], dst_ref=wob.at[dst_slot],
                send_sem=osend.at[idx], recv_sem=orecv.at[idx],
                device_id=(dst_dev,), device_id_type=pl.DeviceIdType.MESH)
            rq.start()
            ro.start()
            return rq, ro

        a_r = pair_copy(right, 0, 1, 0)
        a_l = pair_copy(left, 0, 2, 1)
        compute_group(0, me)
        for r in (*a_r, *a_l):
            r.wait()
        b_r = pair_copy(right, 1, 3, 2)
        compute_group(1, left)
        compute_group(2, right)
        for r in b_r:
            r.wait()
        compute_group(3, (me + 2) % N_DEV)

        for b in range(B):
            out_ref[b] = accs[b]

    return pl.pallas_call(
        body,
        out_shape=jax.ShapeDtypeStruct((B, SQ, D_MODEL), jnp.float32),
        in_specs=[pl.BlockSpec(memory_space=pltpu.VMEM)] * 5,
        out_specs=pl.BlockSpec(memory_space=pltpu.VMEM),
        scratch_shapes=[
            pltpu.VMEM((N_DEV, D_MODEL, G_COLS), jnp.bfloat16),
            pltpu.VMEM((N_DEV, G_COLS, D_MODEL), jnp.bfloat16),
            pltpu.SemaphoreType.DMA((N_DEV - 1,)),
            pltpu.SemaphoreType.DMA((N_DEV - 1,)),
            pltpu.SemaphoreType.DMA((N_DEV - 1,)),
            pltpu.SemaphoreType.DMA((N_DEV - 1,)),
            pltpu.SemaphoreType.DMA((2,)),
        ],
        compiler_params=pltpu.CompilerParams(collective_id=0),
    )(x, Wq, K2, V2, Wo)


# device time: 63030 ns/iter; 1.6660x vs baseline; 1.0461x over previous
import jax
import jax.numpy as jnp
from jax import lax
from jax.experimental import pallas as pl
from jax.experimental.pallas import tpu as pltpu

N_DEV = 4
B = 2
SQ = 512
SKV = 512
HQ = 32
DH = 64
HG = HQ // N_DEV
D_MODEL = 768
D_HEADS = HQ * DH
G_COLS = HG * DH
HALF_COLS = G_COLS // 2


def kernel(x, Wq, K_ext, V_ext, Wo):
    bf = jnp.bfloat16
    K2 = K_ext.reshape(B, SKV, D_HEADS).astype(bf)
    V2 = V_ext.reshape(B, SKV, D_HEADS).astype(bf)
    x = x.astype(bf)
    Wq = Wq.astype(bf)
    Wo = Wo.astype(bf)

    def body(x_ref, wq_ref, k_ref, v_ref, wo_ref, out_ref,
             wqb, wob, qsend, qrecv, osend, orecv):
        me = lax.axis_index("i")
        left = (me - 1) % N_DEV
        right = (me + 1) % N_DEV

        bsem = pltpu.get_barrier_semaphore()
        pl.semaphore_signal(bsem, inc=1, device_id=(left,),
                            device_id_type=pl.DeviceIdType.MESH)
        pl.semaphore_signal(bsem, inc=1, device_id=(right,),
                            device_id_type=pl.DeviceIdType.MESH)
        pl.semaphore_wait(bsem, 2)

        wqb[0] = wq_ref[:, :HALF_COLS]
        wqb[1] = wq_ref[:, HALF_COLS:]
        wob[0] = wo_ref[:HALF_COLS, :]
        wob[1] = wo_ref[HALF_COLS:, :]

        li = lax.broadcasted_iota(jnp.int32, (SQ, SKV), 0)
        kj = lax.broadcasted_iota(jnp.int32, (SQ, SKV), 1)
        mask = (li // 64) % 4 == (kj // 64) % 4

        xs = [x_ref[b] for b in range(B)]
        accs = [jnp.zeros((SQ, D_MODEL), jnp.float32) for _ in range(B)]

        def compute_half(slot, origin, half):
            col0 = origin * G_COLS + half * HALF_COLS
            for b in range(B):
                q = lax.dot_general(
                    xs[b], wqb[slot], (((1,), (0,)), ((), ())),
                    preferred_element_type=jnp.float32)
                q = (q * 0.125).astype(jnp.bfloat16)
                kg = k_ref[b, :, pl.ds(col0, HALF_COLS)]
                vg = v_ref[b, :, pl.ds(col0, HALF_COLS)]
                ctx_parts = []
                for hh in range(HG // 2):
                    qh = q[:, hh * DH:(hh + 1) * DH]
                    kh = kg[:, hh * DH:(hh + 1) * DH]
                    vh = vg[:, hh * DH:(hh + 1) * DH]
                    s = lax.dot_general(
                        qh, kh, (((1,), (1,)), ((), ())),
                        preferred_element_type=jnp.float32)
                    e = jnp.where(mask, jnp.exp(s), 0.0)
                    w = (e / jnp.sum(e, axis=1, keepdims=True)).astype(
                        jnp.bfloat16)
                    ctx_parts.append(lax.dot_general(
                        w, vh, (((1,), (0,)), ((), ())),
                        preferred_element_type=jnp.float32).astype(
                            jnp.bfloat16))
                ctx = jnp.concatenate(ctx_parts, axis=1)
                accs[b] = accs[b] + lax.dot_general(
                    ctx, wob[slot], (((1,), (0,)), ((), ())),
                    preferred_element_type=jnp.float32)

        def pair_copy(dst_dev, src_slot, dst_slot, idx):
            rq = pltpu.make_async_remote_copy(
                src_ref=wqb.at[src_slot], dst_ref=wqb.at[dst_slot],
                send_sem=qsend.at[idx], recv_sem=qrecv.at[idx],
                device_id=(dst_dev,), device_id_type=pl.DeviceIdType.MESH)
            ro = pltpu.make_async_remote_copy(
                src_ref=wob.at[src_slot], dst_ref=wob.at[dst_slot],
                send_sem=osend.at[idx], recv_sem=orecv.at[idx],
                device_id=(dst_dev,), device_id_type=pl.DeviceIdType.MESH)
            rq.start()
            ro.start()
            return rq, ro

        a_r0 = pair_copy(right, 0, 2, 0)
        a_l0 = pair_copy(left, 0, 4, 1)
        a_r1 = pair_copy(right, 1, 3, 2)
        a_l1 = pair_copy(left, 1, 5, 3)
        compute_half(0, me, 0)
        compute_half(1, me, 1)
        for r in (*a_r0, *a_l0):
            r.wait()
        b_0 = pair_copy(right, 2, 6, 4)
        compute_half(2, left, 0)
        for r in (*a_r1, *a_l1):
            r.wait()
        b_1 = pair_copy(right, 3, 7, 5)
        compute_half(3, left, 1)
        compute_half(4, right, 0)
        compute_half(5, right, 1)
        for r in (*b_0, *b_1):
            r.wait()
        diag = (me + 2) % N_DEV
        compute_half(6, diag, 0)
        compute_half(7, diag, 1)

        for b in range(B):
            out_ref[b] = accs[b]

    return pl.pallas_call(
        body,
        out_shape=jax.ShapeDtypeStruct((B, SQ, D_MODEL), jnp.float32),
        in_specs=[pl.BlockSpec(memory_space=pltpu.VMEM)] * 5,
        out_specs=pl.BlockSpec(memory_space=pltpu.VMEM),
        scratch_shapes=[
            pltpu.VMEM((2 * N_DEV, D_MODEL, HALF_COLS), jnp.bfloat16),
            pltpu.VMEM((2 * N_DEV, HALF_COLS, D_MODEL), jnp.bfloat16),
            pltpu.SemaphoreType.DMA((6,)),
            pltpu.SemaphoreType.DMA((6,)),
            pltpu.SemaphoreType.DMA((6,)),
            pltpu.SemaphoreType.DMA((6,)),
        ],
        compiler_params=pltpu.CompilerParams(collective_id=0),
    )(x, Wq, K2, V2, Wo)


# device time: 62366 ns/iter; 1.6837x vs baseline; 1.0106x over previous
import jax
import jax.numpy as jnp
from jax import lax
from jax.experimental import pallas as pl
from jax.experimental.pallas import tpu as pltpu

N_DEV = 4
B = 2
SQ = 512
SKV = 512
HQ = 32
DH = 64
HG = HQ // N_DEV
D_MODEL = 768
D_HEADS = HQ * DH
G_COLS = HG * DH
HALF_COLS = G_COLS // 2


def kernel(x, Wq, K_ext, V_ext, Wo):
    bf = jnp.bfloat16
    K2 = K_ext.reshape(B, SKV, D_HEADS).astype(bf)
    V2 = V_ext.reshape(B, SKV, D_HEADS).astype(bf)
    x = x.astype(bf)
    Wq = Wq.astype(bf)
    Wo = Wo.astype(bf)

    def body(x_ref, wq_ref, k_ref, v_ref, wo_ref, out_ref,
             wqb, wob, qsend, qrecv, osend, orecv):
        me = lax.axis_index("i")
        left = (me - 1) % N_DEV
        right = (me + 1) % N_DEV

        bsem = pltpu.get_barrier_semaphore()
        pl.semaphore_signal(bsem, inc=1, device_id=(left,),
                            device_id_type=pl.DeviceIdType.MESH)
        pl.semaphore_signal(bsem, inc=1, device_id=(right,),
                            device_id_type=pl.DeviceIdType.MESH)
        pl.semaphore_wait(bsem, 2)

        wqb[0] = wq_ref[:, :HALF_COLS]
        wqb[1] = wq_ref[:, HALF_COLS:]
        wob[0] = wo_ref[:HALF_COLS, :]
        wob[1] = wo_ref[HALF_COLS:, :]

        li = lax.broadcasted_iota(jnp.int32, (SQ, SKV), 0)
        kj = lax.broadcasted_iota(jnp.int32, (SQ, SKV), 1)
        mask = (li // 64) % 4 == (kj // 64) % 4

        xcat = x_ref[:].reshape(B * SQ, D_MODEL)
        acc = [jnp.zeros((B * SQ, D_MODEL), jnp.float32)]

        def compute_half(slot, origin, half):
            col0 = origin * G_COLS + half * HALF_COLS
            q = lax.dot_general(
                xcat, wqb[slot], (((1,), (0,)), ((), ())),
                preferred_element_type=jnp.float32)
            q = (q * 0.125).astype(jnp.bfloat16)
            ctx_parts = []
            for b in range(B):
                kg = k_ref[b, :, pl.ds(col0, HALF_COLS)]
                vg = v_ref[b, :, pl.ds(col0, HALF_COLS)]
                for hh in range(HG // 2):
                    qh = q[b * SQ:(b + 1) * SQ, hh * DH:(hh + 1) * DH]
                    kh = kg[:, hh * DH:(hh + 1) * DH]
                    vh = vg[:, hh * DH:(hh + 1) * DH]
                    s = lax.dot_general(
                        qh, kh, (((1,), (1,)), ((), ())),
                        preferred_element_type=jnp.float32)
                    e = jnp.where(mask, jnp.exp(s), 0.0)
                    w = (e * (1.0 / jnp.sum(e, axis=1, keepdims=True))
                         ).astype(jnp.bfloat16)
                    ctx_parts.append(lax.dot_general(
                        w, vh, (((1,), (0,)), ((), ())),
                        preferred_element_type=jnp.float32).astype(
                            jnp.bfloat16))
            ctx = jnp.concatenate(
                [jnp.concatenate(ctx_parts[b * (HG // 2):
                                           (b + 1) * (HG // 2)], axis=1)
                 for b in range(B)], axis=0)
            acc[0] = acc[0] + lax.dot_general(
                ctx, wob[slot], (((1,), (0,)), ((), ())),
                preferred_element_type=jnp.float32)

        def pair_copy(dst_dev, src_slot, dst_slot, idx):
            rq = pltpu.make_async_remote_copy(
                src_ref=wqb.at[src_slot], dst_ref=wqb.at[dst_slot],
                send_sem=qsend.at[idx], recv_sem=qrecv.at[idx],
                device_id=(dst_dev,), device_id_type=pl.DeviceIdType.MESH)
            ro = pltpu.make_async_remote_copy(
                src_ref=wob.at[src_slot], dst_ref=wob.at[dst_slot],
                send_sem=osend.at[idx], recv_sem=orecv.at[idx],
                device_id=(dst_dev,), device_id_type=pl.DeviceIdType.MESH)
            rq.start()
            ro.start()
            return rq, ro

        a_r0 = pair_copy(right, 0, 2, 0)
        a_l0 = pair_copy(left, 0, 4, 1)
        a_r1 = pair_copy(right, 1, 3, 2)
        a_l1 = pair_copy(left, 1, 5, 3)
        compute_half(0, me, 0)
        compute_half(1, me, 1)
        for r in (*a_r0, *a_l0):
            r.wait()
        b_0 = pair_copy(right, 2, 6, 4)
        compute_half(2, left, 0)
        for r in (*a_r1, *a_l1):
            r.wait()
        b_1 = pair_copy(right, 3, 7, 5)
        compute_half(3, left, 1)
        compute_half(4, right, 0)
        compute_half(5, right, 1)
        for r in (*b_0, *b_1):
            r.wait()
        diag = (me + 2) % N_DEV
        compute_half(6, diag, 0)
        compute_half(7, diag, 1)

        out_ref[:] = acc[0].reshape(B, SQ, D_MODEL)

    return pl.pallas_call(
        body,
        out_shape=jax.ShapeDtypeStruct((B, SQ, D_MODEL), jnp.float32),
        in_specs=[pl.BlockSpec(memory_space=pltpu.VMEM)] * 5,
        out_specs=pl.BlockSpec(memory_space=pltpu.VMEM),
        scratch_shapes=[
            pltpu.VMEM((2 * N_DEV, D_MODEL, HALF_COLS), jnp.bfloat16),
            pltpu.VMEM((2 * N_DEV, HALF_COLS, D_MODEL), jnp.bfloat16),
            pltpu.SemaphoreType.DMA((6,)),
            pltpu.SemaphoreType.DMA((6,)),
            pltpu.SemaphoreType.DMA((6,)),
            pltpu.SemaphoreType.DMA((6,)),
        ],
        compiler_params=pltpu.CompilerParams(collective_id=0),
    )(x, Wq, K2, V2, Wo)


# device time: 58264 ns/iter; 1.8022x vs baseline; 1.0704x over previous
import jax
import jax.numpy as jnp
from jax import lax
from jax.experimental import pallas as pl
from jax.experimental.pallas import tpu as pltpu

N_DEV = 4
B = 2
SQ = 512
SKV = 512
HQ = 32
DH = 64
HG = HQ // N_DEV
D_MODEL = 768
D_HEADS = HQ * DH
G_COLS = HG * DH
HALF_COLS = G_COLS // 2


def kernel(x, Wq, K_ext, V_ext, Wo):
    bf = jnp.bfloat16
    K2 = K_ext.reshape(B, SKV, D_HEADS).astype(bf)
    V2 = V_ext.reshape(B, SKV, D_HEADS).astype(bf)
    x = x.astype(bf)
    Wq = Wq.astype(bf)
    Wo = Wo.astype(bf)

    def body(x_ref, wq_ref, k_ref, v_ref, wo_ref, out_ref,
             wqb, wob, ctxb, qsend, qrecv, osend, orecv, csem):
        me = lax.axis_index("i")
        left = (me - 1) % N_DEV
        right = (me + 1) % N_DEV

        bsem = pltpu.get_barrier_semaphore()
        pl.semaphore_signal(bsem, inc=1, device_id=(left,),
                            device_id_type=pl.DeviceIdType.MESH)
        pl.semaphore_signal(bsem, inc=1, device_id=(right,),
                            device_id_type=pl.DeviceIdType.MESH)
        pl.semaphore_wait(bsem, 2)

        wqb[0] = wq_ref[:, :HALF_COLS]
        wqb[1] = wq_ref[:, HALF_COLS:]
        c0 = pltpu.make_async_copy(
            wo_ref.at[pl.ds(0, HALF_COLS)], wob.at[2 * me], csem.at[0])
        c1 = pltpu.make_async_copy(
            wo_ref.at[pl.ds(HALF_COLS, HALF_COLS)], wob.at[2 * me + 1],
            csem.at[1])
        c0.start()
        c1.start()

        li = lax.broadcasted_iota(jnp.int32, (SQ, SKV), 0)
        kj = lax.broadcasted_iota(jnp.int32, (SQ, SKV), 1)
        mask = (li // 64) % 4 == (kj // 64) % 4

        xcat = x_ref[:].reshape(B * SQ, D_MODEL)

        def compute_half(slot, origin, half):
            col0 = origin * G_COLS + half * HALF_COLS
            q = lax.dot_general(
                xcat, wqb[slot], (((1,), (0,)), ((), ())),
                preferred_element_type=jnp.float32)
            q = (q * 0.125).astype(jnp.bfloat16)
            ctx_parts = []
            for b in range(B):
                kg = k_ref[b, :, pl.ds(col0, HALF_COLS)]
                vg = v_ref[b, :, pl.ds(col0, HALF_COLS)]
                for hh in range(HG // 2):
                    qh = q[b * SQ:(b + 1) * SQ, hh * DH:(hh + 1) * DH]
                    kh = kg[:, hh * DH:(hh + 1) * DH]
                    vh = vg[:, hh * DH:(hh + 1) * DH]
                    s = lax.dot_general(
                        qh, kh, (((1,), (1,)), ((), ())),
                        preferred_element_type=jnp.float32)
                    e = jnp.where(mask, jnp.exp(s), 0.0)
                    w = (e * (1.0 / jnp.sum(e, axis=1, keepdims=True))
                         ).astype(jnp.bfloat16)
                    ctx_parts.append(lax.dot_general(
                        w, vh, (((1,), (0,)), ((), ())),
                        preferred_element_type=jnp.float32).astype(
                            jnp.bfloat16))
            ctx = jnp.concatenate(
                [jnp.concatenate(ctx_parts[b * (HG // 2):
                                           (b + 1) * (HG // 2)], axis=1)
                 for b in range(B)], axis=0)
            ctxb[:, pl.ds(col0, HALF_COLS)] = ctx

        def wq_copy(dst_dev, src_slot, dst_slot, idx):
            r = pltpu.make_async_remote_copy(
                src_ref=wqb.at[src_slot], dst_ref=wqb.at[dst_slot],
                send_sem=qsend.at[idx], recv_sem=qrecv.at[idx],
                device_id=(dst_dev,), device_id_type=pl.DeviceIdType.MESH)
            r.start()
            return r

        def wo_copy(dst_dev, slot, idx):
            r = pltpu.make_async_remote_copy(
                src_ref=wob.at[slot], dst_ref=wob.at[slot],
                send_sem=osend.at[idx], recv_sem=orecv.at[idx],
                device_id=(dst_dev,), device_id_type=pl.DeviceIdType.MESH)
            r.start()
            return r

        aq_r0 = wq_copy(right, 0, 2, 0)
        aq_l0 = wq_copy(left, 0, 4, 1)
        aq_r1 = wq_copy(right, 1, 3, 2)
        aq_l1 = wq_copy(left, 1, 5, 3)
        c0.wait()
        c1.wait()
        compute_half(0, me, 0)
        compute_half(1, me, 1)
        aq_r0.wait()
        aq_l0.wait()
        bq0 = wq_copy(right, 2, 6, 4)
        awo_r0 = wo_copy(right, 2 * me, 0)
        awo_r1 = wo_copy(right, 2 * me + 1, 1)
        awo_l0 = wo_copy(left, 2 * me, 2)
        awo_l1 = wo_copy(left, 2 * me + 1, 3)
        compute_half(2, left, 0)
        aq_r1.wait()
        aq_l1.wait()
        bq1 = wq_copy(right, 3, 7, 5)
        compute_half(3, left, 1)
        compute_half(4, right, 0)
        awo_r0.wait()
        bwo0 = wo_copy(right, 2 * left, 4)
        awo_r1.wait()
        bwo1 = wo_copy(right, 2 * left + 1, 5)
        compute_half(5, right, 1)
        bq0.wait()
        bq1.wait()
        diag = (me + 2) % N_DEV
        compute_half(6, diag, 0)
        compute_half(7, diag, 1)
        awo_l0.wait()
        awo_l1.wait()
        bwo0.wait()
        bwo1.wait()

        wfull = wob[:].reshape(2 * N_DEV * HALF_COLS, D_MODEL)
        out = lax.dot_general(
            ctxb[:], wfull, (((1,), (0,)), ((), ())),
            preferred_element_type=jnp.float32)
        out_ref[:] = out.reshape(B, SQ, D_MODEL)

    return pl.pallas_call(
        body,
        out_shape=jax.ShapeDtypeStruct((B, SQ, D_MODEL), jnp.float32),
        in_specs=[pl.BlockSpec(memory_space=pltpu.VMEM)] * 5,
        out_specs=pl.BlockSpec(memory_space=pltpu.VMEM),
        scratch_shapes=[
            pltpu.VMEM((2 * N_DEV, D_MODEL, HALF_COLS), jnp.bfloat16),
            pltpu.VMEM((2 * N_DEV, HALF_COLS, D_MODEL), jnp.bfloat16),
            pltpu.VMEM((B * SQ, D_HEADS), jnp.bfloat16),
            pltpu.SemaphoreType.DMA((6,)),
            pltpu.SemaphoreType.DMA((6,)),
            pltpu.SemaphoreType.DMA((6,)),
            pltpu.SemaphoreType.DMA((6,)),
            pltpu.SemaphoreType.DMA((2,)),
        ],
        compiler_params=pltpu.CompilerParams(collective_id=0),
    )(x, Wq, K2, V2, Wo)
